# baseline (speedup 1.0000x reference)
import functools

import jax
import jax.numpy as jnp
from jax import lax
from jax.experimental import pallas as pl
from jax.experimental.pallas import tpu as pltpu
from jax.experimental.pallas import tpu_sc as plsc

_NUM_SEEDS = 64
_HIDDEN = 2048
_CHUNK = 32
_NUM_BP = 10
_TOKENS = 16384
_NW = 32
_RPW = _TOKENS // _NW
_R = 32
_NBLK = _RPW // _R


def _sc_body(x_hbm, lc_hbm, bp_hbm, st_hbm, bwr_hbm, out_hbm,
             lc_v, bp_v, st_v, idx_v, w_v, a_v, b_v, xb_v, sem):
    c = lax.axis_index("c")
    s = lax.axis_index("s")
    wid = s * 2 + c

    pltpu.sync_copy(lc_hbm, lc_v)
    pltpu.sync_copy(bp_hbm, bp_v)
    pltpu.sync_copy(st_hbm, st_v)

    for g in range(4):
        sl = pl.ds(g * 16, 16)
        bpv = bp_v[sl]
        seeds = lax.iota(jnp.int32, 16) + g * 16
        idx_v[sl] = jnp.clip(bpv, 0, _NUM_BP - 1) * _NUM_SEEDS + seeds

    pltpu.async_copy(bwr_hbm.at[idx_v], w_v, sem).wait()

    one = jnp.full((16,), 1.0, jnp.float32)
    half = jnp.full((16,), 0.5, jnp.float32)
    zero = jnp.full((16,), 0.0, jnp.float32)
    for g in range(4):
        sl = pl.ds(g * 16, 16)
        lc = lc_v[sl]
        bp = bp_v[sl]
        st = st_v[sl]
        active = (lc >= 2) & (lc <= 5) & (bp < _NUM_BP)
        act0 = active & (st == 0)
        act1 = active & (st == 1)
        actm = active & (st >= 2)
        m1 = jnp.where(act0, zero, jnp.where(act1, one, jnp.where(actm, half, one)))
        m2 = jnp.where(act1, one, jnp.where(actm, half, zero))
        m3 = jnp.where(act0, one, zero)
        for l in range(16):
            sidx = g * 16 + l
            m1s = jnp.broadcast_to(m1[l:l + 1], (16,))
            m2s = jnp.broadcast_to(m2[l:l + 1], (16,))
            m3s = jnp.broadcast_to(m3[l:l + 1], (16,))
            for h in (0, 16):
                wv = w_v[sidx, pl.ds(h, 16)]
                a_v[sidx, pl.ds(h, 16)] = wv * m3s + m1s
                b_v[sidx, pl.ds(h, 16)] = wv * m2s

    def blk_body(blk, carry):
        base = wid * _RPW + blk * _R
        pltpu.sync_copy(x_hbm.at[pl.ds(base, _R)], xb_v)

        def seed_body(sidx, carry2):
            for h in (0, 16):
                col = sidx * _CHUNK + h
                av = a_v[sidx, pl.ds(h, 16)]
                bv = b_v[sidx, pl.ds(h, 16)]

                def row_body(r, carry3):
                    xb_v[r, pl.ds(col, 16)] = xb_v[r, pl.ds(col, 16)] * av + bv
                    return carry3

                lax.fori_loop(0, _R, row_body, 0, unroll=16)
            return carry2

        lax.fori_loop(0, _NUM_SEEDS, seed_body, 0, unroll=2)
        pltpu.sync_copy(xb_v, out_hbm.at[pl.ds(base, _R)])
        return carry

    lax.fori_loop(0, _NBLK, blk_body, 0)


@jax.jit
def kernel(x, lifecycle_states, blueprint_ids, grafting_strategies, blueprint_weights):
    bwr = jnp.pad(blueprint_weights.reshape(_NUM_BP * _NUM_SEEDS, _CHUNK),
                  ((0, 0), (0, 128 - _CHUNK)))
    mesh = plsc.VectorSubcoreMesh(core_axis_name="c", subcore_axis_name="s")
    run = pl.kernel(
        _sc_body,
        out_type=jax.ShapeDtypeStruct((_TOKENS, _HIDDEN), jnp.float32),
        mesh=mesh,
        scratch_types=[
            pltpu.VMEM((_NUM_SEEDS,), jnp.int32),
            pltpu.VMEM((_NUM_SEEDS,), jnp.int32),
            pltpu.VMEM((_NUM_SEEDS,), jnp.int32),
            pltpu.VMEM((_NUM_SEEDS,), jnp.int32),
            pltpu.VMEM((_NUM_SEEDS, 128), jnp.float32),
            pltpu.VMEM((_NUM_SEEDS, _CHUNK), jnp.float32),
            pltpu.VMEM((_NUM_SEEDS, _CHUNK), jnp.float32),
            pltpu.VMEM((_R, _HIDDEN), jnp.float32),
            pltpu.SemaphoreType.DMA,
        ],
    )
    return run(x, lifecycle_states, blueprint_ids, grafting_strategies, bwr)


# speedup vs baseline: 2.0486x; 2.0486x over previous
import functools

import jax
import jax.numpy as jnp
from jax.experimental import pallas as pl

_NUM_SEEDS = 64
_HIDDEN = 2048
_CHUNK = _HIDDEN // _NUM_SEEDS
_NUM_BP = 10
_BT = 1024


def _combine_kernel(lc_ref, bp_ref, st_ref, e_ref, bw_ref, x_ref, o_ref):
    lc = lc_ref[...]
    bp = bp_ref[...]
    st = st_ref[...]

    active = (lc >= 2) & (lc <= 5) & (bp < _NUM_BP)
    act0 = active & (st == 0)
    act1 = active & (st == 1)
    actm = active & (st != 0) & (st != 1)

    one = jnp.float32(1.0)
    half = jnp.float32(0.5)
    m1 = jnp.where(act0, 0.0, jnp.where(act1, one, jnp.where(actm, half, one)))
    m2 = jnp.where(act1, one, jnp.where(actm, half, 0.0))
    m3 = jnp.where(act0, one, 0.0)

    bpc = jnp.clip(bp, 0, _NUM_BP - 1)
    rows = [m1.astype(jnp.float32), m2.astype(jnp.float32), m3.astype(jnp.float32)]
    for r in range(_NUM_BP):
        rows.append((bpc == r).astype(jnp.float32))
    p = jnp.concatenate(rows, axis=0)

    q = jnp.dot(p, e_ref[...], preferred_element_type=jnp.float32)
    m1f = q[0:1, :]
    m2f = q[1:2, :]
    m3f = q[2:3, :]
    onehot = q[3:3 + _NUM_BP, :]

    w = jnp.sum(onehot * bw_ref[...], axis=0, keepdims=True)
    a = m1f + w * m3f
    b = w * m2f
    o_ref[...] = x_ref[...] * a + b


@jax.jit
def kernel(x, lifecycle_states, blueprint_ids, grafting_strategies, blueprint_weights):
    tokens = x.shape[0]
    lc = lifecycle_states.reshape(1, _NUM_SEEDS)
    bp = blueprint_ids.reshape(1, _NUM_SEEDS)
    st = grafting_strategies.reshape(1, _NUM_SEEDS)
    e = (jnp.arange(_HIDDEN, dtype=jnp.int32)[None, :] // _CHUNK
         == jnp.arange(_NUM_SEEDS, dtype=jnp.int32)[:, None]).astype(jnp.float32)

    grid = (tokens // _BT,)
    small = lambda i: (0, 0)
    return pl.pallas_call(
        _combine_kernel,
        grid=grid,
        in_specs=[
            pl.BlockSpec((1, _NUM_SEEDS), small),
            pl.BlockSpec((1, _NUM_SEEDS), small),
            pl.BlockSpec((1, _NUM_SEEDS), small),
            pl.BlockSpec((_NUM_SEEDS, _HIDDEN), small),
            pl.BlockSpec((_NUM_BP, _HIDDEN), small),
            pl.BlockSpec((_BT, _HIDDEN), lambda i: (i, 0)),
        ],
        out_specs=pl.BlockSpec((_BT, _HIDDEN), lambda i: (i, 0)),
        out_shape=jax.ShapeDtypeStruct((tokens, _HIDDEN), x.dtype),
    )(lc, bp, st, e, blueprint_weights, x)


# speedup vs baseline: 2.0515x; 1.0014x over previous
import jax
import jax.numpy as jnp
from jax.experimental import pallas as pl

_NUM_SEEDS = 64
_HIDDEN = 2048
_CHUNK = _HIDDEN // _NUM_SEEDS
_NUM_BP = 10
_BT = 1024


def _combine_kernel(lc_ref, bp_ref, st_ref, e_ref, bw_ref, x_ref, o_ref):
    lc = lc_ref[...]
    bp = bp_ref[...]
    st = st_ref[...]

    active = (lc >= 2) & (lc <= 5) & (bp < _NUM_BP)
    act0 = active & (st == 0)
    act1 = active & (st == 1)
    actm = active & (st != 0) & (st != 1)

    one = jnp.float32(1.0)
    half = jnp.float32(0.5)
    m1 = jnp.where(act0, 0.0, jnp.where(act1, one, jnp.where(actm, half, one)))
    m2 = jnp.where(act1, one, jnp.where(actm, half, 0.0))
    m3 = jnp.where(act0, one, 0.0)

    bpc = jnp.clip(bp, 0, _NUM_BP - 1)
    rows = [m1.astype(jnp.float32), m2.astype(jnp.float32), m3.astype(jnp.float32)]
    for r in range(_NUM_BP):
        rows.append((bpc == r).astype(jnp.float32))
    p = jnp.concatenate(rows, axis=0)

    q = jnp.dot(p, e_ref[...], preferred_element_type=jnp.float32)
    m1f = q[0:1, :]
    m2f = q[1:2, :]
    m3f = q[2:3, :]
    onehot = q[3:3 + _NUM_BP, :]

    w = jnp.sum(onehot * bw_ref[...], axis=0, keepdims=True)
    a = m1f + w * m3f
    b = w * m2f
    o_ref[...] = x_ref[...] * a + b


@jax.jit
def kernel(x, lifecycle_states, blueprint_ids, grafting_strategies, blueprint_weights):
    tokens = x.shape[0]
    lc = lifecycle_states.reshape(1, _NUM_SEEDS)
    bp = blueprint_ids.reshape(1, _NUM_SEEDS)
    st = grafting_strategies.reshape(1, _NUM_SEEDS)
    e = (jnp.arange(_HIDDEN, dtype=jnp.int32)[None, :] // _CHUNK
         == jnp.arange(_NUM_SEEDS, dtype=jnp.int32)[:, None]).astype(jnp.float32)

    grid = (tokens // _BT,)
    small = lambda i: (0, 0)
    return pl.pallas_call(
        _combine_kernel,
        grid=grid,
        in_specs=[
            pl.BlockSpec((1, _NUM_SEEDS), small),
            pl.BlockSpec((1, _NUM_SEEDS), small),
            pl.BlockSpec((1, _NUM_SEEDS), small),
            pl.BlockSpec((_NUM_SEEDS, _HIDDEN), small),
            pl.BlockSpec((_NUM_BP, _HIDDEN), small),
            pl.BlockSpec((_BT, _HIDDEN), lambda i: (i, 0)),
        ],
        out_specs=pl.BlockSpec((_BT, _HIDDEN), lambda i: (i, 0)),
        out_shape=jax.ShapeDtypeStruct((tokens, _HIDDEN), x.dtype),
    )(lc, bp, st, e, blueprint_weights, x)
